# baseline (speedup 1.0000x reference)
import functools

import jax
import jax.numpy as jnp
from jax.experimental import pallas as pl
from jax.experimental.pallas import tpu as pltpu

_NCORES = 2


def _worklist_body(a_ref, m_ref, e_ref, s_ref, n_ref, *,
                   bt: int, t: int, e: int, num_work: int):
    a = a_ref[...]
    offs = [jnp.int32(0)]
    for i in range(1, e):
        offs.append(jnp.sum((a < i).astype(jnp.int32)))
    offs.append(jnp.int32(t))

    t_core = t // _NCORES
    tiles_core = t_core // bt
    for c in range(_NCORES):
        lo, hi = c * t_core, (c + 1) * t_core
        tile0 = c * tiles_core
        cum = jnp.int32(0)
        last_e = jnp.int32(e - 1)
        for i in range(e):
            start = jnp.clip(offs[i], lo, hi)
            end = jnp.clip(offs[i + 1], lo, hi)
            size = end - start
            ft = start // bt
            lt = jnp.maximum(end - 1, 0) // bt
            tp = jnp.where(size > 0, lt - ft + 1, 0)

            def body(k, _, cum=cum, ft=ft, start=start, end=end, i=i, c=c):
                w = cum + k
                m = ft + k
                m_ref[c, w] = m
                e_ref[c, w] = jnp.int32(i)
                s_ref[c, w] = jnp.maximum(start, m * bt)
                n_ref[c, w] = jnp.minimum(end, (m + 1) * bt)
                return 0

            jax.lax.fori_loop(0, tp, body, 0)
            cum = cum + tp
            last_e = jnp.where(size > 0, jnp.int32(i), last_e)

        def dummy(w, _, c=c, last_e=last_e):
            m_ref[c, w] = tile0 + tiles_core - 1
            e_ref[c, w] = last_e
            s_ref[c, w] = 0
            n_ref[c, w] = 0
            return 0

        jax.lax.fori_loop(cum, num_work, dummy, 0)


def _gmm_body(m_ref, e_ref, starts_ref, ends_ref,
              x_ref, w_ref, o_ref, *, bt: int):
    c = pl.program_id(0)
    w = pl.program_id(1)
    prev = m_ref[c, jnp.maximum(w - 1, 0)]
    first = jnp.logical_or(w == 0, m_ref[c, w] != prev)
    base = m_ref[c, w] * bt
    rows = base + jax.lax.broadcasted_iota(jnp.int32, (bt, 1), 0)
    mask = jnp.logical_and(rows >= starts_ref[c, w], rows < ends_ref[c, w])
    contrib = jax.lax.dot_general(
        x_ref[...], w_ref[0],
        dimension_numbers=(((1,), (1,)), ((), ())),
        preferred_element_type=jnp.float32)
    contrib = jnp.where(mask, contrib, 0.0)

    @pl.when(first)
    def _():
        o_ref[...] = contrib

    @pl.when(jnp.logical_not(first))
    def _():
        o_ref[...] += contrib


def kernel(input_tokens, weight, expert_assignments):
    t, d_in = input_tokens.shape
    e, d_out, _ = weight.shape

    bt = 256
    num_work = t // (_NCORES * bt) + e - 1

    a2d = expert_assignments.astype(jnp.int32).reshape(t // 128, 128)
    wl_shape = jax.ShapeDtypeStruct((_NCORES, num_work), jnp.int32)
    m_w, e_w, starts, ends = pl.pallas_call(
        functools.partial(_worklist_body, bt=bt, t=t, e=e,
                          num_work=num_work),
        in_specs=[pl.BlockSpec((t // 128, 128), lambda: (0, 0))],
        out_specs=[pl.BlockSpec(memory_space=pltpu.SMEM)] * 4,
        out_shape=[wl_shape] * 4,
    )(a2d)

    grid_spec = pltpu.PrefetchScalarGridSpec(
        num_scalar_prefetch=4,
        grid=(_NCORES, num_work),
        in_specs=[
            pl.BlockSpec((bt, d_in),
                         lambda c, w, mt, ex, st, en: (mt[c, w], 0)),
            pl.BlockSpec((1, d_out, d_in),
                         lambda c, w, mt, ex, st, en: (ex[c, w], 0, 0)),
        ],
        out_specs=pl.BlockSpec((bt, d_out),
                               lambda c, w, mt, ex, st, en: (mt[c, w], 0)),
    )

    out = pl.pallas_call(
        functools.partial(_gmm_body, bt=bt),
        grid_spec=grid_spec,
        out_shape=jax.ShapeDtypeStruct((t, d_out), jnp.float32),
        compiler_params=pltpu.CompilerParams(
            dimension_semantics=("parallel", "arbitrary")),
    )(m_w, e_w, starts, ends, input_tokens, weight)
    return out


# speedup vs baseline: 1.1996x; 1.1996x over previous
import functools

import jax
import jax.numpy as jnp
from jax.experimental import pallas as pl
from jax.experimental.pallas import tpu as pltpu


def _worklist_body(a_ref, m_ref, e_ref, slot_ref, nxt_ref, s_ref, n_ref, *,
                   bt: int, t: int, e: int, num_work: int):
    a = a_ref[...]
    m_tiles_total = t // bt
    offs = [jnp.int32(0)]
    for i in range(1, e):
        offs.append(jnp.sum((a < i).astype(jnp.int32)))
    offs.append(jnp.int32(t))

    cum = jnp.int32(0)
    last_e = jnp.int32(e - 1)
    slot = jnp.int32(0)
    started = jnp.int32(0)
    for i in range(e):
        start, end = offs[i], offs[i + 1]
        size = end - start
        ft = start // bt
        lt = jnp.maximum(end - 1, 0) // bt
        tp = jnp.where(size > 0, lt - ft + 1, 0)
        slot = jnp.where(tp > 0, jnp.where(started > 0, 1 - slot, slot), slot)
        started = jnp.where(tp > 0, 1, started)

        def body(k, _, cum=cum, ft=ft, start=start, end=end, i=i, slot=slot):
            w = cum + k
            m = ft + k
            m_ref[w] = m
            e_ref[w] = jnp.int32(i)
            slot_ref[w] = slot
            s_ref[w] = jnp.maximum(start, m * bt)
            n_ref[w] = jnp.minimum(end, (m + 1) * bt)
            return 0

        jax.lax.fori_loop(0, tp, body, 0)
        cum = cum + tp
        last_e = jnp.where(size > 0, jnp.int32(i), last_e)
    final_slot = slot

    def dummy(w, _):
        m_ref[w] = m_tiles_total - 1
        e_ref[w] = last_e
        slot_ref[w] = final_slot
        s_ref[w] = 0
        n_ref[w] = 0
        return 0

    jax.lax.fori_loop(cum, num_work, dummy, 0)

    def back(k, following):
        w = num_work - 1 - k
        cur = e_ref[w]
        is_first = jnp.logical_or(w == 0, e_ref[jnp.maximum(w - 1, 0)] != cur)
        nxt_ref[w] = jnp.where(is_first, following, -1)
        return jnp.where(is_first, cur, following)

    jax.lax.fori_loop(0, num_work, back, jnp.int32(-1))


def _gmm_body(m_ref, e_ref, slot_ref, nxt_ref, starts_ref, ends_ref,
              x_ref, w_hbm, o_ref, w_ring, dma_sems, *, bt: int):
    w = pl.program_id(0)
    prev_m = m_ref[jnp.maximum(w - 1, 0)]
    first_m = jnp.logical_or(w == 0, m_ref[w] != prev_m)
    slot = slot_ref[w]
    prev_slot = slot_ref[jnp.maximum(w - 1, 0)]
    first_run = jnp.logical_or(w == 0, slot != prev_slot)

    @pl.when(w == 0)
    def _():
        pltpu.make_async_copy(w_hbm.at[e_ref[0]], w_ring.at[slot_ref[0]],
                              dma_sems.at[slot_ref[0]]).start()

    nxt = nxt_ref[w]

    @pl.when(nxt >= 0)
    def _():
        pltpu.make_async_copy(w_hbm.at[nxt], w_ring.at[1 - slot],
                              dma_sems.at[1 - slot]).start()

    @pl.when(first_run)
    def _():
        pltpu.make_async_copy(w_hbm.at[e_ref[w]], w_ring.at[slot],
                              dma_sems.at[slot]).wait()

    base = m_ref[w] * bt
    rows = base + jax.lax.broadcasted_iota(jnp.int32, (bt, 1), 0)
    mask = jnp.logical_and(rows >= starts_ref[w], rows < ends_ref[w])
    contrib = jax.lax.dot_general(
        x_ref[...], w_ring[slot],
        dimension_numbers=(((1,), (1,)), ((), ())),
        preferred_element_type=jnp.float32)
    contrib = jnp.where(mask, contrib, 0.0)

    @pl.when(first_m)
    def _():
        o_ref[...] = contrib

    @pl.when(jnp.logical_not(first_m))
    def _():
        o_ref[...] += contrib


def kernel(input_tokens, weight, expert_assignments):
    t, d_in = input_tokens.shape
    e, d_out, _ = weight.shape

    bt = 256
    num_work = t // bt + e - 1

    a2d = expert_assignments.astype(jnp.int32).reshape(t // 128, 128)
    wl_shape = jax.ShapeDtypeStruct((num_work,), jnp.int32)
    m_w, e_w, slot_w, nxt_w, starts, ends = pl.pallas_call(
        functools.partial(_worklist_body, bt=bt, t=t, e=e,
                          num_work=num_work),
        in_specs=[pl.BlockSpec((t // 128, 128), lambda: (0, 0))],
        out_specs=[pl.BlockSpec(memory_space=pltpu.SMEM)] * 6,
        out_shape=[wl_shape] * 6,
    )(a2d)

    grid_spec = pltpu.PrefetchScalarGridSpec(
        num_scalar_prefetch=6,
        grid=(num_work,),
        in_specs=[
            pl.BlockSpec((bt, d_in),
                         lambda w, mt, ex, sl, nx, st, en: (mt[w], 0)),
            pl.BlockSpec(memory_space=pl.ANY),
        ],
        out_specs=pl.BlockSpec((bt, d_out),
                               lambda w, mt, ex, sl, nx, st, en: (mt[w], 0)),
        scratch_shapes=[
            pltpu.VMEM((2, d_out, d_in), jnp.float32),
            pltpu.SemaphoreType.DMA((2,)),
        ],
    )

    out = pl.pallas_call(
        functools.partial(_gmm_body, bt=bt),
        grid_spec=grid_spec,
        out_shape=jax.ShapeDtypeStruct((t, d_out), jnp.float32),
        compiler_params=pltpu.CompilerParams(
            dimension_semantics=("arbitrary",)),
    )(m_w, e_w, slot_w, nxt_w, starts, ends, input_tokens, weight)
    return out
